# Initial kernel scaffold; baseline (speedup 1.0000x reference)
import jax
import jax.numpy as jnp
from jax.experimental import pallas as pl


def kernel(nodes, edges, senders, receivers, W_node, W_edge):
    raise NotImplementedError("write your pallas kernel here")


# speedup vs baseline: 4.4123x; 4.4123x over previous
import functools

import jax
import jax.numpy as jnp
from jax import lax
from jax.experimental import pallas as pl
from jax.experimental.pallas import tpu as pltpu
from jax.experimental.pallas import tpu_sc as plsc

N = 10000
E = 320000
EOFF = E - N
C = 128

NC = 2
NS = 16
NW = NC * NS

CH = 2000
KCH = CH // 16
F32 = jnp.float32


def _mesh():
    return plsc.VectorSubcoreMesh(core_axis_name="c", subcore_axis_name="s")


def _wid():
    return lax.axis_index("s") * NC + lax.axis_index("c")


def _unpack_idx(pk):
    snd = jnp.bitwise_and(pk, jnp.int32(0xFFFF))
    rcv = lax.shift_right_logical(pk, jnp.int32(16))
    return snd, rcv


def _sc_pass_a(edges, nodes, packed):
    cpw = C // NW
    nch = E // CH

    def body(edges_hbm, nodes_hbm, packed_hbm, sent_hbm, nodes_t, acc, ebuf, ibuf):
        c0 = _wid() * cpw
        pltpu.sync_copy(nodes_hbm.at[pl.ds(c0, cpw)], nodes_t)

        @pl.loop(0, N // 16)
        def _(i):
            off = i * 16
            for j in range(cpw):
                acc[j, pl.ds(off, 16)] = jnp.zeros((16,), F32)

        @pl.loop(0, nch)
        def _(g):
            col = g * CH
            pltpu.sync_copy(packed_hbm.at[pl.ds(col, CH)], ibuf)
            pltpu.sync_copy(edges_hbm.at[pl.ds(c0, cpw), pl.ds(col, CH)], ebuf)

            @pl.loop(0, KCH)
            def _(k):
                off = k * 16
                snd, rcv = _unpack_idx(ibuf[pl.ds(off, 16)])
                for j in range(cpw):
                    gath = plsc.load_gather(nodes_t.at[j], [rcv])
                    e = ebuf[j, pl.ds(off, 16)]
                    plsc.addupdate_scatter(acc.at[j], [snd], e * gath)

        pltpu.sync_copy(acc, sent_hbm.at[pl.ds(c0, cpw)])

    return pl.kernel(
        body,
        out_type=jax.ShapeDtypeStruct((C, N), F32),
        mesh=_mesh(),
        compiler_params=pltpu.CompilerParams(use_tc_tiling_on_sc=False, needs_layout_passes=False),
        scratch_types=[
            pltpu.VMEM((cpw, N), F32),
            pltpu.VMEM((cpw, N), F32),
            pltpu.VMEM((cpw, CH), F32),
            pltpu.VMEM((CH,), jnp.int32),
        ],
    )(edges, nodes, packed)


def _sc_pass_a2(b1, edges0, nodes1, p11, p21, packed):
    cpg = 2
    ngroups = C // cpg
    reps = ngroups // NW
    nch = EOFF // CH

    def body(b1_hbm, e0_hbm, n1_hbm, p11_hbm, p21_hbm, packed_hbm, sent_hbm,
             n1_t, p11_t, p21_t, acc, bbuf, ibuf):
        w = _wid()
        for rep in range(reps):
            c0 = (w + rep * NW) * cpg
            pltpu.sync_copy(n1_hbm.at[pl.ds(c0, cpg)], n1_t)
            pltpu.sync_copy(p11_hbm.at[pl.ds(c0, cpg)], p11_t)
            pltpu.sync_copy(p21_hbm.at[pl.ds(c0, cpg)], p21_t)
            pltpu.sync_copy(e0_hbm.at[pl.ds(c0, cpg), pl.ds(0, N)], acc)

            @pl.loop(0, N // 16)
            def _(i):
                off = i * 16
                for j in range(cpg):
                    acc[j, pl.ds(off, 16)] = (
                        acc[j, pl.ds(off, 16)] * n1_t[j, pl.ds(off, 16)])

            @pl.loop(0, nch)
            def _(g):
                col = N + g * CH
                pltpu.sync_copy(packed_hbm.at[pl.ds(col, CH)], ibuf)
                pltpu.sync_copy(b1_hbm.at[pl.ds(c0, cpg), pl.ds(col, CH)], bbuf)

                @pl.loop(0, KCH)
                def _(k):
                    off = k * 16
                    snd, rcv = _unpack_idx(ibuf[pl.ds(off, 16)])
                    for j in range(cpg):
                        e1 = (bbuf[j, pl.ds(off, 16)]
                              + plsc.load_gather(p11_t.at[j], [snd])
                              + plsc.load_gather(p21_t.at[j], [rcv]))
                        gath = plsc.load_gather(n1_t.at[j], [rcv])
                        plsc.addupdate_scatter(acc.at[j], [snd], e1 * gath)

            pltpu.sync_copy(acc, sent_hbm.at[pl.ds(c0, cpg)])

    return pl.kernel(
        body,
        out_type=jax.ShapeDtypeStruct((C, N), F32),
        mesh=_mesh(),
        compiler_params=pltpu.CompilerParams(use_tc_tiling_on_sc=False, needs_layout_passes=False),
        scratch_types=[
            pltpu.VMEM((cpg, N), F32),
            pltpu.VMEM((cpg, N), F32),
            pltpu.VMEM((cpg, N), F32),
            pltpu.VMEM((cpg, N), F32),
            pltpu.VMEM((cpg, CH), F32),
            pltpu.VMEM((CH,), jnp.int32),
        ],
    )(b1, edges0, nodes1, p11, p21, packed)


def _sc_pass_b(b2base, edges0, r1, r2, packed):
    cpw = C // NW
    nch = EOFF // CH
    ndch = N // CH

    def body(b2_hbm, e0_hbm, r1_hbm, r2_hbm, packed_hbm, out_hbm,
             r1_t, r2_t, bbuf, obuf, ibuf):
        c0 = _wid() * cpw
        pltpu.sync_copy(r1_hbm.at[pl.ds(c0, cpw)], r1_t)
        pltpu.sync_copy(r2_hbm.at[pl.ds(c0, cpw)], r2_t)

        @pl.loop(0, ndch)
        def _(g):
            col = g * CH
            pltpu.sync_copy(e0_hbm.at[pl.ds(c0, cpw), pl.ds(col, CH)], obuf)
            pltpu.sync_copy(obuf, out_hbm.at[pl.ds(c0, cpw), pl.ds(col, CH)])

        @pl.loop(0, nch)
        def _(g):
            col = N + g * CH
            pltpu.sync_copy(packed_hbm.at[pl.ds(col, CH)], ibuf)
            pltpu.sync_copy(b2_hbm.at[pl.ds(c0, cpw), pl.ds(col, CH)], bbuf)

            @pl.loop(0, KCH)
            def _(k):
                off = k * 16
                snd, rcv = _unpack_idx(ibuf[pl.ds(off, 16)])
                for j in range(cpw):
                    obuf[j, pl.ds(off, 16)] = (
                        bbuf[j, pl.ds(off, 16)]
                        + plsc.load_gather(r1_t.at[j], [snd])
                        + plsc.load_gather(r2_t.at[j], [rcv]))

            pltpu.sync_copy(obuf, out_hbm.at[pl.ds(c0, cpw), pl.ds(col, CH)])

    return pl.kernel(
        body,
        out_type=jax.ShapeDtypeStruct((C, E), F32),
        mesh=_mesh(),
        compiler_params=pltpu.CompilerParams(use_tc_tiling_on_sc=False, needs_layout_passes=False),
        scratch_types=[
            pltpu.VMEM((cpw, N), F32),
            pltpu.VMEM((cpw, N), F32),
            pltpu.VMEM((cpw, CH), F32),
            pltpu.VMEM((cpw, CH), F32),
            pltpu.VMEM((CH,), jnp.int32),
        ],
    )(b2base, edges0, r1, r2, packed)


def _tc_big(edges, w_edge):
    EB = 2560
    grid = E // EB

    def body(e_ref, we_ref, b1_ref, b2_ref):
        w1 = we_ref[:, :C]
        x = e_ref[...]
        w1sq = jnp.dot(w1, w1, preferred_element_type=F32)
        b1_ref[...] = jnp.dot(w1, x, preferred_element_type=F32)
        b2_ref[...] = jnp.dot(w1sq, x, preferred_element_type=F32)

    return pl.pallas_call(
        body,
        grid=(grid,),
        in_specs=[
            pl.BlockSpec((C, EB), lambda i: (0, i)),
            pl.BlockSpec((C, 3 * C), lambda i: (0, 0)),
        ],
        out_specs=[pl.BlockSpec((C, EB), lambda i: (0, i))] * 2,
        out_shape=[jax.ShapeDtypeStruct((C, E), F32)] * 2,
    )(edges, w_edge)


def _tc_node_update1(nodes, sent, w_node, w_edge):
    def body(n_ref, s_ref, wn_ref, we_ref, nn_ref, p1_ref, p2_ref):
        nn = (jnp.dot(wn_ref[:, :C], n_ref[...], preferred_element_type=F32)
              + jnp.dot(wn_ref[:, C:], s_ref[...], preferred_element_type=F32))
        nn_ref[...] = nn
        p1_ref[...] = jnp.dot(we_ref[:, C:2 * C], nn, preferred_element_type=F32)
        p2_ref[...] = jnp.dot(we_ref[:, 2 * C:], nn, preferred_element_type=F32)

    return pl.pallas_call(
        body,
        out_shape=[jax.ShapeDtypeStruct((C, N), F32)] * 3,
    )(nodes, sent, w_node, w_edge)


def _tc_node_update2(nodes1, sent2, p11, p21, w_node, w_edge):
    def body(n_ref, s_ref, p11_ref, p21_ref, wn_ref, we_ref,
             nn_ref, r1_ref, r2_ref):
        w1 = we_ref[:, :C]
        nn = (jnp.dot(wn_ref[:, :C], n_ref[...], preferred_element_type=F32)
              + jnp.dot(wn_ref[:, C:], s_ref[...], preferred_element_type=F32))
        nn_ref[...] = nn
        r1_ref[...] = (jnp.dot(w1, p11_ref[...], preferred_element_type=F32)
                       + jnp.dot(we_ref[:, C:2 * C], nn, preferred_element_type=F32))
        r2_ref[...] = (jnp.dot(w1, p21_ref[...], preferred_element_type=F32)
                       + jnp.dot(we_ref[:, 2 * C:], nn, preferred_element_type=F32))

    return pl.pallas_call(
        body,
        out_shape=[jax.ShapeDtypeStruct((C, N), F32)] * 3,
    )(nodes1, sent2, p11, p21, w_node, w_edge)


def kernel(nodes, edges, senders, receivers, W_node, W_edge):
    packed = jnp.bitwise_or(senders.astype(jnp.int32),
                            lax.shift_left(receivers.astype(jnp.int32), 16))

    b1, b2base = _tc_big(edges, W_edge)
    sent1 = _sc_pass_a(edges, nodes, packed)
    nodes1, p11, p21 = _tc_node_update1(nodes, sent1, W_node, W_edge)
    sent2 = _sc_pass_a2(b1, edges, nodes1, p11, p21, packed)
    nodes2, r1, r2 = _tc_node_update2(nodes1, sent2, p11, p21, W_node, W_edge)
    edges2 = _sc_pass_b(b2base, edges, r1, r2, packed)
    return (nodes2, edges2, senders, receivers)


# speedup vs baseline: 6.1174x; 1.3864x over previous
import jax
import jax.numpy as jnp
from jax import lax
from jax.experimental import pallas as pl
from jax.experimental.pallas import tpu as pltpu
from jax.experimental.pallas import tpu_sc as plsc

N = 10000
E = 320000
EOFF = E - N
C = 128

NC = 2
NS = 16
NW = NC * NS

F32 = jnp.float32
I32 = jnp.int32

_SC_PARAMS = pltpu.CompilerParams(
    use_tc_tiling_on_sc=False, needs_layout_passes=False)


def _mesh():
    return plsc.VectorSubcoreMesh(core_axis_name="c", subcore_axis_name="s")


def _wid():
    return lax.axis_index("s") * NC + lax.axis_index("c")


def _unpack_idx(pk):
    snd = jnp.bitwise_and(pk, jnp.int32(0xFFFF))
    rcv = lax.shift_right_logical(pk, jnp.int32(16))
    return snd, rcv


def _pipelined_chunks(nch, start, work):
    start(0, 0)
    start(1, 1)

    @pl.loop(0, (nch + 1) // 2)
    def _(p):
        for b in range(2):
            g = p * 2 + b

            @pl.when(g < nch)
            def _():
                work(g, b)

                @pl.when(g + 2 < nch)
                def _():
                    start(g + 2, b)


def _sc_pass_a(edges, nodes, packed):
    cpw = C // NW
    CH = 4000
    nch = E // CH
    KCH = CH // 16

    def body(edges_hbm, nodes_hbm, packed_hbm, sent_hbm,
             nodes_t, acc, ebuf, ibuf, isem, esem):
        c0 = _wid() * cpw
        pltpu.sync_copy(nodes_hbm.at[pl.ds(c0, cpw)], nodes_t)

        @pl.loop(0, N // 16, unroll=8)
        def _(i):
            off = i * 16
            for j in range(cpw):
                acc[j, pl.ds(off, 16)] = jnp.zeros((16,), F32)

        def start(g, b):
            col = g * CH
            pltpu.async_copy(packed_hbm.at[pl.ds(col, CH)], ibuf.at[b],
                             isem.at[b])
            pltpu.async_copy(edges_hbm.at[pl.ds(c0, cpw), pl.ds(col, CH)],
                             ebuf.at[b], esem.at[b])

        def work(g, b):
            pltpu.make_async_copy(packed_hbm.at[pl.ds(0, CH)], ibuf.at[b],
                                  isem.at[b]).wait()
            pltpu.make_async_copy(edges_hbm.at[pl.ds(0, cpw), pl.ds(0, CH)],
                                  ebuf.at[b], esem.at[b]).wait()

            @pl.loop(0, KCH, unroll=5)
            def _(k):
                off = k * 16
                snd, rcv = _unpack_idx(ibuf[b, pl.ds(off, 16)])
                for j in range(cpw):
                    gath = plsc.load_gather(nodes_t.at[j], [rcv])
                    e = ebuf[b, j, pl.ds(off, 16)]
                    plsc.addupdate_scatter(acc.at[j], [snd], e * gath)

        _pipelined_chunks(nch, start, work)
        pltpu.sync_copy(acc, sent_hbm.at[pl.ds(c0, cpw)])

    return pl.kernel(
        body,
        out_type=jax.ShapeDtypeStruct((C, N), F32),
        mesh=_mesh(),
        compiler_params=_SC_PARAMS,
        scratch_types=[
            pltpu.VMEM((cpw, N), F32),
            pltpu.VMEM((cpw, N), F32),
            pltpu.VMEM((2, cpw, CH), F32),
            pltpu.VMEM((2, CH), I32),
            pltpu.SemaphoreType.DMA((2,)),
            pltpu.SemaphoreType.DMA((2,)),
        ],
    )(edges, nodes, packed)


def _sc_pass_a2(b1, edges0, nodes1, p11, p21, packed):
    cpg = 2
    reps = C // cpg // NW
    CH = 2000
    nch = EOFF // CH
    KCH = CH // 16

    def body(b1_hbm, e0_hbm, n1_hbm, p11_hbm, p21_hbm, packed_hbm, sent_hbm,
             n1_t, p11_t, p21_t, acc, bbuf, ibuf, isem, esem):
        w = _wid()
        for rep in range(reps):
            c0 = (w + rep * NW) * cpg
            pltpu.sync_copy(n1_hbm.at[pl.ds(c0, cpg)], n1_t)
            pltpu.sync_copy(p11_hbm.at[pl.ds(c0, cpg)], p11_t)
            pltpu.sync_copy(p21_hbm.at[pl.ds(c0, cpg)], p21_t)
            pltpu.sync_copy(e0_hbm.at[pl.ds(c0, cpg), pl.ds(0, N)], acc)

            @pl.loop(0, N // 16, unroll=8)
            def _(i):
                off = i * 16
                for j in range(cpg):
                    acc[j, pl.ds(off, 16)] = (
                        acc[j, pl.ds(off, 16)] * n1_t[j, pl.ds(off, 16)])

            def start(g, b):
                col = N + g * CH
                pltpu.async_copy(packed_hbm.at[pl.ds(col, CH)], ibuf.at[b],
                                 isem.at[b])
                pltpu.async_copy(b1_hbm.at[pl.ds(c0, cpg), pl.ds(col, CH)],
                                 bbuf.at[b], esem.at[b])

            def work(g, b):
                pltpu.make_async_copy(packed_hbm.at[pl.ds(0, CH)], ibuf.at[b],
                                      isem.at[b]).wait()
                pltpu.make_async_copy(
                    b1_hbm.at[pl.ds(0, cpg), pl.ds(0, CH)], bbuf.at[b],
                    esem.at[b]).wait()

                @pl.loop(0, KCH, unroll=5)
                def _(k):
                    off = k * 16
                    snd, rcv = _unpack_idx(ibuf[b, pl.ds(off, 16)])
                    for j in range(cpg):
                        e1 = (bbuf[b, j, pl.ds(off, 16)]
                              + plsc.load_gather(p11_t.at[j], [snd])
                              + plsc.load_gather(p21_t.at[j], [rcv]))
                        gath = plsc.load_gather(n1_t.at[j], [rcv])
                        plsc.addupdate_scatter(acc.at[j], [snd], e1 * gath)

            _pipelined_chunks(nch, start, work)
            pltpu.sync_copy(acc, sent_hbm.at[pl.ds(c0, cpg)])

    return pl.kernel(
        body,
        out_type=jax.ShapeDtypeStruct((C, N), F32),
        mesh=_mesh(),
        compiler_params=_SC_PARAMS,
        scratch_types=[
            pltpu.VMEM((cpg, N), F32),
            pltpu.VMEM((cpg, N), F32),
            pltpu.VMEM((cpg, N), F32),
            pltpu.VMEM((cpg, N), F32),
            pltpu.VMEM((2, cpg, CH), F32),
            pltpu.VMEM((2, CH), I32),
            pltpu.SemaphoreType.DMA((2,)),
            pltpu.SemaphoreType.DMA((2,)),
        ],
    )(b1, edges0, nodes1, p11, p21, packed)


def _sc_pass_b(b2base, edges0, r1, r2, packed):
    cpw = C // NW
    CH = 2000
    nch = EOFF // CH
    ndch = N // CH
    KCH = CH // 16

    def body(b2_hbm, e0_hbm, r1_hbm, r2_hbm, packed_hbm, out_hbm,
             r1_t, r2_t, bbuf, obuf, ibuf, isem, esem, osem):
        c0 = _wid() * cpw
        pltpu.sync_copy(r1_hbm.at[pl.ds(c0, cpw)], r1_t)
        pltpu.sync_copy(r2_hbm.at[pl.ds(c0, cpw)], r2_t)

        @pl.loop(0, ndch)
        def _(g):
            col = g * CH
            pltpu.sync_copy(e0_hbm.at[pl.ds(c0, cpw), pl.ds(col, CH)],
                            obuf.at[0])
            pltpu.sync_copy(obuf.at[0], out_hbm.at[pl.ds(c0, cpw),
                                                   pl.ds(col, CH)])

        def start(g, b):
            col = N + g * CH
            pltpu.async_copy(packed_hbm.at[pl.ds(col, CH)], ibuf.at[b],
                             isem.at[b])
            pltpu.async_copy(b2_hbm.at[pl.ds(c0, cpw), pl.ds(col, CH)],
                             bbuf.at[b], esem.at[b])

        def work(g, b):
            pltpu.make_async_copy(packed_hbm.at[pl.ds(0, CH)], ibuf.at[b],
                                  isem.at[b]).wait()
            pltpu.make_async_copy(b2_hbm.at[pl.ds(0, cpw), pl.ds(0, CH)],
                                  bbuf.at[b], esem.at[b]).wait()

            @pl.when(g >= 2)
            def _():
                pltpu.make_async_copy(
                    obuf.at[b], out_hbm.at[pl.ds(0, cpw), pl.ds(0, CH)],
                    osem.at[b]).wait()

            @pl.loop(0, KCH, unroll=5)
            def _(k):
                off = k * 16
                snd, rcv = _unpack_idx(ibuf[b, pl.ds(off, 16)])
                for j in range(cpw):
                    obuf[b, j, pl.ds(off, 16)] = (
                        bbuf[b, j, pl.ds(off, 16)]
                        + plsc.load_gather(r1_t.at[j], [snd])
                        + plsc.load_gather(r2_t.at[j], [rcv]))

            col = N + g * CH
            pltpu.async_copy(obuf.at[b],
                             out_hbm.at[pl.ds(c0, cpw), pl.ds(col, CH)],
                             osem.at[b])

        _pipelined_chunks(nch, start, work)
        for b in range(2):
            pltpu.make_async_copy(obuf.at[b],
                                  out_hbm.at[pl.ds(0, cpw), pl.ds(0, CH)],
                                  osem.at[b]).wait()

    return pl.kernel(
        body,
        out_type=jax.ShapeDtypeStruct((C, E), F32),
        mesh=_mesh(),
        compiler_params=_SC_PARAMS,
        scratch_types=[
            pltpu.VMEM((cpw, N), F32),
            pltpu.VMEM((cpw, N), F32),
            pltpu.VMEM((2, cpw, CH), F32),
            pltpu.VMEM((2, cpw, CH), F32),
            pltpu.VMEM((2, CH), I32),
            pltpu.SemaphoreType.DMA((2,)),
            pltpu.SemaphoreType.DMA((2,)),
            pltpu.SemaphoreType.DMA((2,)),
        ],
    )(b2base, edges0, r1, r2, packed)


def _tc_big(edges, w_edge):
    EB = 2560
    grid = E // EB

    def body(e_ref, we_ref, b1_ref, b2_ref):
        w1 = we_ref[:, :C]
        x = e_ref[...]
        w1sq = jnp.dot(w1, w1, preferred_element_type=F32)
        b1_ref[...] = jnp.dot(w1, x, preferred_element_type=F32)
        b2_ref[...] = jnp.dot(w1sq, x, preferred_element_type=F32)

    return pl.pallas_call(
        body,
        grid=(grid,),
        in_specs=[
            pl.BlockSpec((C, EB), lambda i: (0, i)),
            pl.BlockSpec((C, 3 * C), lambda i: (0, 0)),
        ],
        out_specs=[pl.BlockSpec((C, EB), lambda i: (0, i))] * 2,
        out_shape=[jax.ShapeDtypeStruct((C, E), F32)] * 2,
    )(edges, w_edge)


def _tc_node_update1(nodes, sent, w_node, w_edge):
    def body(n_ref, s_ref, wn_ref, we_ref, nn_ref, p1_ref, p2_ref):
        nn = (jnp.dot(wn_ref[:, :C], n_ref[...], preferred_element_type=F32)
              + jnp.dot(wn_ref[:, C:], s_ref[...], preferred_element_type=F32))
        nn_ref[...] = nn
        p1_ref[...] = jnp.dot(we_ref[:, C:2 * C], nn, preferred_element_type=F32)
        p2_ref[...] = jnp.dot(we_ref[:, 2 * C:], nn, preferred_element_type=F32)

    return pl.pallas_call(
        body,
        out_shape=[jax.ShapeDtypeStruct((C, N), F32)] * 3,
    )(nodes, sent, w_node, w_edge)


def _tc_node_update2(nodes1, sent2, p11, p21, w_node, w_edge):
    def body(n_ref, s_ref, p11_ref, p21_ref, wn_ref, we_ref,
             nn_ref, r1_ref, r2_ref):
        w1 = we_ref[:, :C]
        nn = (jnp.dot(wn_ref[:, :C], n_ref[...], preferred_element_type=F32)
              + jnp.dot(wn_ref[:, C:], s_ref[...], preferred_element_type=F32))
        nn_ref[...] = nn
        r1_ref[...] = (jnp.dot(w1, p11_ref[...], preferred_element_type=F32)
                       + jnp.dot(we_ref[:, C:2 * C], nn, preferred_element_type=F32))
        r2_ref[...] = (jnp.dot(w1, p21_ref[...], preferred_element_type=F32)
                       + jnp.dot(we_ref[:, 2 * C:], nn, preferred_element_type=F32))

    return pl.pallas_call(
        body,
        out_shape=[jax.ShapeDtypeStruct((C, N), F32)] * 3,
    )(nodes1, sent2, p11, p21, w_node, w_edge)


def kernel(nodes, edges, senders, receivers, W_node, W_edge):
    packed = jnp.bitwise_or(senders.astype(jnp.int32),
                            lax.shift_left(receivers.astype(jnp.int32), 16))

    b1, b2base = _tc_big(edges, W_edge)
    sent1 = _sc_pass_a(edges, nodes, packed)
    nodes1, p11, p21 = _tc_node_update1(nodes, sent1, W_node, W_edge)
    sent2 = _sc_pass_a2(b1, edges, nodes1, p11, p21, packed)
    nodes2, r1, r2 = _tc_node_update2(nodes1, sent2, p11, p21, W_node, W_edge)
    edges2 = _sc_pass_b(b2base, edges, r1, r2, packed)
    return (nodes2, edges2, senders, receivers)


# speedup vs baseline: 11.3575x; 1.8566x over previous
import jax
import jax.numpy as jnp
from jax import lax
from jax.experimental import pallas as pl
from jax.experimental.pallas import tpu as pltpu
from jax.experimental.pallas import tpu_sc as plsc

N = 10000
E = 320000
EOFF = E - N
C = 128

NC = 2
NS = 16
NW = NC * NS

F32 = jnp.float32
I32 = jnp.int32

_SC_PARAMS = pltpu.CompilerParams(
    use_tc_tiling_on_sc=False, needs_layout_passes=False)


def _mesh():
    return plsc.VectorSubcoreMesh(core_axis_name="c", subcore_axis_name="s")


def _wid():
    return lax.axis_index("s") * NC + lax.axis_index("c")


def _unpack_idx(pk):
    snd = jnp.bitwise_and(pk, jnp.int32(0xFFFF))
    rcv = lax.shift_right_logical(pk, jnp.int32(16))
    return snd, rcv


def _pipelined_chunks(nch, start, work):
    start(0, 0)
    start(1, 1)

    @pl.loop(0, (nch + 1) // 2)
    def _(p):
        for b in range(2):
            g = p * 2 + b

            @pl.when(g < nch)
            def _():
                work(g, b)

                @pl.when(g + 2 < nch)
                def _():
                    start(g + 2, b)


def _sc_pass_a(edges, nodes, packed):
    cpw = C // NW
    CH = 4000
    nch = E // CH
    KCH = CH // 16

    def body(edges_hbm, nodes_hbm, packed_hbm, sent_hbm,
             nodes_t, acc, ebuf, ibuf, isem, esem):
        c0 = _wid() * cpw
        pltpu.sync_copy(nodes_hbm.at[pl.ds(c0, cpw)], nodes_t)

        @pl.loop(0, N // 16, unroll=8)
        def _(i):
            off = i * 16
            for j in range(cpw):
                acc[j, pl.ds(off, 16)] = jnp.zeros((16,), F32)

        def start(g, b):
            col = g * CH
            pltpu.async_copy(packed_hbm.at[pl.ds(col, CH)], ibuf.at[b],
                             isem.at[b])
            pltpu.async_copy(edges_hbm.at[pl.ds(c0, cpw), pl.ds(col, CH)],
                             ebuf.at[b], esem.at[b])

        def work(g, b):
            pltpu.make_async_copy(packed_hbm.at[pl.ds(0, CH)], ibuf.at[b],
                                  isem.at[b]).wait()
            pltpu.make_async_copy(edges_hbm.at[pl.ds(0, cpw), pl.ds(0, CH)],
                                  ebuf.at[b], esem.at[b]).wait()

            @plsc.parallel_loop(0, KCH, unroll=5)
            def _(k):
                off = k * 16
                snd, rcv = _unpack_idx(ibuf[b, pl.ds(off, 16)])
                for j in range(cpw):
                    gath = plsc.load_gather(nodes_t.at[j], [rcv])
                    e = ebuf[b, j, pl.ds(off, 16)]
                    plsc.addupdate_scatter(acc.at[j], [snd], e * gath)

        _pipelined_chunks(nch, start, work)
        pltpu.sync_copy(acc, sent_hbm.at[pl.ds(c0, cpw)])

    return pl.kernel(
        body,
        out_type=jax.ShapeDtypeStruct((C, N), F32),
        mesh=_mesh(),
        compiler_params=_SC_PARAMS,
        scratch_types=[
            pltpu.VMEM((cpw, N), F32),
            pltpu.VMEM((cpw, N), F32),
            pltpu.VMEM((2, cpw, CH), F32),
            pltpu.VMEM((2, CH), I32),
            pltpu.SemaphoreType.DMA((2,)),
            pltpu.SemaphoreType.DMA((2,)),
        ],
    )(edges, nodes, packed)


def _sc_pass_a2(b1, edges0, nodes1, p11, p21, packed):
    cpg = 2
    reps = C // cpg // NW
    CH = 2000
    nch = EOFF // CH
    KCH = CH // 16

    def body(b1_hbm, e0_hbm, n1_hbm, p11_hbm, p21_hbm, packed_hbm, sent_hbm,
             n1_t, p11_t, p21_t, acc, bbuf, ibuf, isem, esem):
        w = _wid()
        for rep in range(reps):
            c0 = (w + rep * NW) * cpg
            pltpu.sync_copy(n1_hbm.at[pl.ds(c0, cpg)], n1_t)
            pltpu.sync_copy(p11_hbm.at[pl.ds(c0, cpg)], p11_t)
            pltpu.sync_copy(p21_hbm.at[pl.ds(c0, cpg)], p21_t)
            pltpu.sync_copy(e0_hbm.at[pl.ds(c0, cpg), pl.ds(0, N)], acc)

            @pl.loop(0, N // 16, unroll=8)
            def _(i):
                off = i * 16
                for j in range(cpg):
                    acc[j, pl.ds(off, 16)] = (
                        acc[j, pl.ds(off, 16)] * n1_t[j, pl.ds(off, 16)])

            def start(g, b):
                col = N + g * CH
                pltpu.async_copy(packed_hbm.at[pl.ds(col, CH)], ibuf.at[b],
                                 isem.at[b])
                pltpu.async_copy(b1_hbm.at[pl.ds(c0, cpg), pl.ds(col, CH)],
                                 bbuf.at[b], esem.at[b])

            def work(g, b):
                pltpu.make_async_copy(packed_hbm.at[pl.ds(0, CH)], ibuf.at[b],
                                      isem.at[b]).wait()
                pltpu.make_async_copy(
                    b1_hbm.at[pl.ds(0, cpg), pl.ds(0, CH)], bbuf.at[b],
                    esem.at[b]).wait()

                @plsc.parallel_loop(0, KCH, unroll=5)
                def _(k):
                    off = k * 16
                    snd, rcv = _unpack_idx(ibuf[b, pl.ds(off, 16)])
                    for j in range(cpg):
                        e1 = (bbuf[b, j, pl.ds(off, 16)]
                              + plsc.load_gather(p11_t.at[j], [snd])
                              + plsc.load_gather(p21_t.at[j], [rcv]))
                        gath = plsc.load_gather(n1_t.at[j], [rcv])
                        plsc.addupdate_scatter(acc.at[j], [snd], e1 * gath)

            _pipelined_chunks(nch, start, work)
            pltpu.sync_copy(acc, sent_hbm.at[pl.ds(c0, cpg)])

    return pl.kernel(
        body,
        out_type=jax.ShapeDtypeStruct((C, N), F32),
        mesh=_mesh(),
        compiler_params=_SC_PARAMS,
        scratch_types=[
            pltpu.VMEM((cpg, N), F32),
            pltpu.VMEM((cpg, N), F32),
            pltpu.VMEM((cpg, N), F32),
            pltpu.VMEM((cpg, N), F32),
            pltpu.VMEM((2, cpg, CH), F32),
            pltpu.VMEM((2, CH), I32),
            pltpu.SemaphoreType.DMA((2,)),
            pltpu.SemaphoreType.DMA((2,)),
        ],
    )(b1, edges0, nodes1, p11, p21, packed)


def _sc_pass_b(b2base, edges0, r1, r2, packed):
    cpw = C // NW
    CH = 2000
    nch = EOFF // CH
    ndch = N // CH
    KCH = CH // 16

    def body(b2_hbm, e0_hbm, r1_hbm, r2_hbm, packed_hbm, out_hbm,
             r1_t, r2_t, bbuf, obuf, ibuf, isem, esem, osem):
        c0 = _wid() * cpw
        pltpu.sync_copy(r1_hbm.at[pl.ds(c0, cpw)], r1_t)
        pltpu.sync_copy(r2_hbm.at[pl.ds(c0, cpw)], r2_t)

        @pl.loop(0, ndch)
        def _(g):
            col = g * CH
            pltpu.sync_copy(e0_hbm.at[pl.ds(c0, cpw), pl.ds(col, CH)],
                            obuf.at[0])
            pltpu.sync_copy(obuf.at[0], out_hbm.at[pl.ds(c0, cpw),
                                                   pl.ds(col, CH)])

        def start(g, b):
            col = N + g * CH
            pltpu.async_copy(packed_hbm.at[pl.ds(col, CH)], ibuf.at[b],
                             isem.at[b])
            pltpu.async_copy(b2_hbm.at[pl.ds(c0, cpw), pl.ds(col, CH)],
                             bbuf.at[b], esem.at[b])

        def work(g, b):
            pltpu.make_async_copy(packed_hbm.at[pl.ds(0, CH)], ibuf.at[b],
                                  isem.at[b]).wait()
            pltpu.make_async_copy(b2_hbm.at[pl.ds(0, cpw), pl.ds(0, CH)],
                                  bbuf.at[b], esem.at[b]).wait()

            @pl.when(g >= 2)
            def _():
                pltpu.make_async_copy(
                    obuf.at[b], out_hbm.at[pl.ds(0, cpw), pl.ds(0, CH)],
                    osem.at[b]).wait()

            @plsc.parallel_loop(0, KCH, unroll=5)
            def _(k):
                off = k * 16
                snd, rcv = _unpack_idx(ibuf[b, pl.ds(off, 16)])
                for j in range(cpw):
                    obuf[b, j, pl.ds(off, 16)] = (
                        bbuf[b, j, pl.ds(off, 16)]
                        + plsc.load_gather(r1_t.at[j], [snd])
                        + plsc.load_gather(r2_t.at[j], [rcv]))

            col = N + g * CH
            pltpu.async_copy(obuf.at[b],
                             out_hbm.at[pl.ds(c0, cpw), pl.ds(col, CH)],
                             osem.at[b])

        _pipelined_chunks(nch, start, work)
        for b in range(2):
            pltpu.make_async_copy(obuf.at[b],
                                  out_hbm.at[pl.ds(0, cpw), pl.ds(0, CH)],
                                  osem.at[b]).wait()

    return pl.kernel(
        body,
        out_type=jax.ShapeDtypeStruct((C, E), F32),
        mesh=_mesh(),
        compiler_params=_SC_PARAMS,
        scratch_types=[
            pltpu.VMEM((cpw, N), F32),
            pltpu.VMEM((cpw, N), F32),
            pltpu.VMEM((2, cpw, CH), F32),
            pltpu.VMEM((2, cpw, CH), F32),
            pltpu.VMEM((2, CH), I32),
            pltpu.SemaphoreType.DMA((2,)),
            pltpu.SemaphoreType.DMA((2,)),
            pltpu.SemaphoreType.DMA((2,)),
        ],
    )(b2base, edges0, r1, r2, packed)


def _tc_big(edges, w_edge):
    EB = 2560
    grid = E // EB

    def body(e_ref, we_ref, b1_ref, b2_ref):
        w1 = we_ref[:, :C]
        x = e_ref[...]
        w1sq = jnp.dot(w1, w1, preferred_element_type=F32)
        b1_ref[...] = jnp.dot(w1, x, preferred_element_type=F32)
        b2_ref[...] = jnp.dot(w1sq, x, preferred_element_type=F32)

    return pl.pallas_call(
        body,
        grid=(grid,),
        in_specs=[
            pl.BlockSpec((C, EB), lambda i: (0, i)),
            pl.BlockSpec((C, 3 * C), lambda i: (0, 0)),
        ],
        out_specs=[pl.BlockSpec((C, EB), lambda i: (0, i))] * 2,
        out_shape=[jax.ShapeDtypeStruct((C, E), F32)] * 2,
    )(edges, w_edge)


def _tc_node_update1(nodes, sent, w_node, w_edge):
    def body(n_ref, s_ref, wn_ref, we_ref, nn_ref, p1_ref, p2_ref):
        nn = (jnp.dot(wn_ref[:, :C], n_ref[...], preferred_element_type=F32)
              + jnp.dot(wn_ref[:, C:], s_ref[...], preferred_element_type=F32))
        nn_ref[...] = nn
        p1_ref[...] = jnp.dot(we_ref[:, C:2 * C], nn, preferred_element_type=F32)
        p2_ref[...] = jnp.dot(we_ref[:, 2 * C:], nn, preferred_element_type=F32)

    return pl.pallas_call(
        body,
        out_shape=[jax.ShapeDtypeStruct((C, N), F32)] * 3,
    )(nodes, sent, w_node, w_edge)


def _tc_node_update2(nodes1, sent2, p11, p21, w_node, w_edge):
    def body(n_ref, s_ref, p11_ref, p21_ref, wn_ref, we_ref,
             nn_ref, r1_ref, r2_ref):
        w1 = we_ref[:, :C]
        nn = (jnp.dot(wn_ref[:, :C], n_ref[...], preferred_element_type=F32)
              + jnp.dot(wn_ref[:, C:], s_ref[...], preferred_element_type=F32))
        nn_ref[...] = nn
        r1_ref[...] = (jnp.dot(w1, p11_ref[...], preferred_element_type=F32)
                       + jnp.dot(we_ref[:, C:2 * C], nn, preferred_element_type=F32))
        r2_ref[...] = (jnp.dot(w1, p21_ref[...], preferred_element_type=F32)
                       + jnp.dot(we_ref[:, 2 * C:], nn, preferred_element_type=F32))

    return pl.pallas_call(
        body,
        out_shape=[jax.ShapeDtypeStruct((C, N), F32)] * 3,
    )(nodes1, sent2, p11, p21, w_node, w_edge)


def kernel(nodes, edges, senders, receivers, W_node, W_edge):
    packed = jnp.bitwise_or(senders.astype(jnp.int32),
                            lax.shift_left(receivers.astype(jnp.int32), 16))

    b1, b2base = _tc_big(edges, W_edge)
    sent1 = _sc_pass_a(edges, nodes, packed)
    nodes1, p11, p21 = _tc_node_update1(nodes, sent1, W_node, W_edge)
    sent2 = _sc_pass_a2(b1, edges, nodes1, p11, p21, packed)
    nodes2, r1, r2 = _tc_node_update2(nodes1, sent2, p11, p21, W_node, W_edge)
    edges2 = _sc_pass_b(b2base, edges, r1, r2, packed)
    return (nodes2, edges2, senders, receivers)
